# initial kernel scaffold (unmeasured)
import jax
import jax.numpy as jnp
from jax import lax
from jax.experimental import pallas as pl
from jax.experimental.pallas import tpu as pltpu

N_DEV = 4


def kernel(x, w_mat):
    m_per, k = x.shape
    n = w_mat.shape[1]

    def body(x_ref, w_ref, out_ref, wbf_ref, comm_ref, send_sems, recv_sems):
        my = lax.axis_index("i")
        left = lax.rem(my + N_DEV - 1, N_DEV)
        right = lax.rem(my + 1, N_DEV)

        barrier = pltpu.get_barrier_semaphore()
        for nbr in (left, right):
            pl.semaphore_signal(
                barrier, inc=1,
                device_id=(nbr,), device_id_type=pl.DeviceIdType.MESH,
            )
        pl.semaphore_wait(barrier, 2)

        wbf_ref[...] = w_ref[...].astype(jnp.bfloat16)
        comm_ref[0] = x_ref[...].astype(jnp.bfloat16)

        def block(origin, slot):
            y = jnp.dot(
                comm_ref[slot], wbf_ref[...],
                preferred_element_type=jnp.float32,
            )
            out_ref[pl.ds(origin * m_per, m_per), :] = jnp.maximum(y, 0.0)

        block(my, 0)

        for h in range(N_DEV - 1):
            rdma = pltpu.make_async_remote_copy(
                src_ref=comm_ref.at[h],
                dst_ref=comm_ref.at[h + 1],
                send_sem=send_sems.at[h],
                recv_sem=recv_sems.at[h + 1],
                device_id=(right,),
                device_id_type=pl.DeviceIdType.MESH,
            )
            rdma.start()
            rdma.wait()
            origin = lax.rem(my + N_DEV - 1 - h, N_DEV)
            block(origin, h + 1)

    return pl.pallas_call(
        body,
        out_shape=jax.ShapeDtypeStruct((N_DEV * m_per, n), jnp.float32),
        in_specs=[
            pl.BlockSpec(memory_space=pltpu.VMEM),
            pl.BlockSpec(memory_space=pltpu.VMEM),
        ],
        out_specs=pl.BlockSpec(memory_space=pltpu.VMEM),
        scratch_shapes=[
            pltpu.VMEM((k, n), jnp.bfloat16),
            pltpu.VMEM((N_DEV, m_per, k), jnp.bfloat16),
            pltpu.SemaphoreType.DMA((N_DEV,)),
            pltpu.SemaphoreType.DMA((N_DEV,)),
        ],
        compiler_params=pltpu.CompilerParams(collective_id=0),
    )(x, w_mat)


# baseline (device time: 47130 ns/iter reference)
import jax
import jax.numpy as jnp
from jax import lax
from jax.experimental import pallas as pl
from jax.experimental.pallas import tpu as pltpu

N_DEV = 4


def kernel(x, w_mat):
    m_per, k = x.shape
    n = w_mat.shape[1]
    half = m_per // 2

    def body(x_ref, w_ref, out_ref, wbf_ref, comm_ref, send_sems, recv_sems):
        my = lax.axis_index("i")
        left = lax.rem(my + N_DEV - 1, N_DEV)
        right = lax.rem(my + 1, N_DEV)
        opp = lax.rem(my + 2, N_DEV)

        barrier = pltpu.get_barrier_semaphore()
        for nbr in (left, right):
            pl.semaphore_signal(
                barrier, inc=1,
                device_id=(nbr,), device_id_type=pl.DeviceIdType.MESH,
            )
        pl.semaphore_wait(barrier, 2)

        lo = pl.ds(0, half)
        hi = pl.ds(half, half)

        def mk(src_slot, src_half, dst_slot, dst_half, sem, target):
            return pltpu.make_async_remote_copy(
                src_ref=comm_ref.at[src_slot, src_half],
                dst_ref=comm_ref.at[dst_slot, dst_half],
                send_sem=send_sems.at[sem], recv_sem=recv_sems.at[sem],
                device_id=(target,), device_id_type=pl.DeviceIdType.MESH,
            )

        rdma_a0 = mk(0, lo, 2, lo, 0, left)
        rdma_a1 = mk(0, hi, 2, hi, 1, left)
        rdma_c = mk(2, lo, 3, lo, 2, left)
        rdma_b1 = mk(0, hi, 1, hi, 3, right)
        rdma_b0 = mk(0, lo, 1, lo, 4, right)
        rdma_d = mk(1, hi, 3, hi, 5, right)

        comm_ref[0, lo] = x_ref[lo].astype(jnp.bfloat16)
        rdma_a0.start()
        rdma_b0.start()
        comm_ref[0, hi] = x_ref[hi].astype(jnp.bfloat16)
        rdma_a1.start()
        rdma_b1.start()

        wbf_ref[...] = w_ref[...].astype(jnp.bfloat16)

        def block(origin, slot, h_idx):
            y = jnp.dot(
                comm_ref[slot, pl.ds(h_idx * half, half)], wbf_ref[...],
                preferred_element_type=jnp.float32,
            )
            out_ref[pl.ds(origin * m_per + h_idx * half, half), :] = (
                jnp.maximum(y, 0.0)
            )

        block(my, 0, 0)
        block(my, 0, 1)

        rdma_a0.wait_recv()
        rdma_c.start()
        rdma_b1.wait_recv()
        rdma_d.start()
        block(right, 2, 0)
        block(left, 1, 1)

        rdma_a1.wait_recv()
        rdma_b0.wait_recv()
        block(right, 2, 1)
        block(left, 1, 0)

        rdma_c.wait_recv()
        block(opp, 3, 0)
        rdma_d.wait_recv()
        block(opp, 3, 1)

        for r in (rdma_a0, rdma_a1, rdma_b0, rdma_b1, rdma_c, rdma_d):
            r.wait_send()

    return pl.pallas_call(
        body,
        out_shape=jax.ShapeDtypeStruct((N_DEV * m_per, n), jnp.float32),
        in_specs=[
            pl.BlockSpec(memory_space=pltpu.VMEM),
            pl.BlockSpec(memory_space=pltpu.VMEM),
        ],
        out_specs=pl.BlockSpec(memory_space=pltpu.VMEM),
        scratch_shapes=[
            pltpu.VMEM((k, n), jnp.bfloat16),
            pltpu.VMEM((N_DEV, m_per, k), jnp.bfloat16),
            pltpu.SemaphoreType.DMA((6,)),
            pltpu.SemaphoreType.DMA((6,)),
        ],
        compiler_params=pltpu.CompilerParams(collective_id=0),
    )(x, w_mat)
